# baseline (speedup 1.0000x reference)
import jax
import jax.numpy as jnp
from jax import lax
from jax.experimental import pallas as pl
from jax.experimental.pallas import tpu as pltpu
from jax.experimental.pallas import tpu_sc as plsc

_NC = 1
_NS = 16
_L = 16
_NW = _NC * _NS


def _make_qnorm_body(nq, f):
  def _qnorm_body(x_hbm, tab_hbm, out_hbm, xv, tv, ov, sv, sem):
    sid = lax.axis_index("s")
    npt = xv.shape[0]
    base = sid * npt
    inv_step = 1.0 / (nq - 1)

    xcopy = pltpu.async_copy(x_hbm.at[pl.ds(base, npt)], xv, sem)
    @pl.when(sid == 0)
    def _():
        pltpu.sync_copy(tab_hbm, sv)
    plsc.subcore_barrier()
    pltpu.sync_copy(sv, tv)
    xcopy.wait()

    @plsc.parallel_loop(0, npt // _L, 1, unroll=4)
    def body(i):
        off = i * _L
        x16 = xv[pl.ds(off, _L)]
        lane = lax.iota(jnp.int32, _L)
        tix = (base + off + lane) % f
        h = tix + (nq // 2 - 1) * f
        k = nq // 2
        while k >= 2:
            v = plsc.load_gather(tv, [h])
            h = h + jnp.where(v <= x16, (k // 2) * f, -((k // 2) * f))
            k //= 2
        v = plsc.load_gather(tv, [h])
        e = h + jnp.where(v <= x16, 0, -f)
        gl = jnp.minimum(jnp.maximum(e, tix), tix + (nq - 2) * f)
        low = plsc.load_gather(tv, [gl])
        high = plsc.load_gather(tv, [gl + f])
        r = (gl - tix).astype(jnp.float32) * (1.0 / f)
        t = (x16 - low) / (high - low + 1e-9)
        ov[pl.ds(off, _L)] = (r + t) * inv_step

    pltpu.sync_copy(ov, out_hbm.at[pl.ds(base, npt)])

  return _qnorm_body


def kernel(x, q_values, quantiles):
    del quantiles
    b, f = x.shape
    nq = q_values.shape[0]
    n = b * f
    npt = n // _NW
    xf = x.reshape(-1)
    tab = q_values.reshape(-1)
    mesh = plsc.VectorSubcoreMesh(core_axis_name="c", subcore_axis_name="s",
                                  num_cores=_NC)
    out = pl.kernel(
        _make_qnorm_body(nq, f),
        out_type=jax.ShapeDtypeStruct((n,), jnp.float32),
        mesh=mesh,
        compiler_params=pltpu.CompilerParams(needs_layout_passes=False),
        scratch_types=[
            pltpu.VMEM((npt,), jnp.float32),
            pltpu.VMEM((f * nq,), jnp.float32),
            pltpu.VMEM((npt,), jnp.float32),
            pltpu.VMEM_SHARED((f * nq,), jnp.float32),
            pltpu.SemaphoreType.DMA,
        ],
    )(xf, tab)
    return out.reshape(b, f)
